# baseline (speedup 1.0000x reference)
import jax
import jax.numpy as jnp
from jax.experimental import pallas as pl
from jax.experimental.pallas import tpu as pltpu

_CD = jnp.bfloat16
_NT = (((1,), (1,)), ((), ()))


def _make_body(R, L):
    def body(*refs):
        x_hbm = refs[0]
        adj_refs = refs[1:1 + R]
        rel_hbm, w0_hbm, b0_hbm, w1_hbm, b1_hbm = refs[1 + R:6 + R]
        out_ref = refs[6 + R]
        (x_s, rel_s, w0_s, b0_s, w1_s, b1_s,
         wx0_s, wx1_s, relp0_s, relp1_s, sems) = refs[7 + R:]
        b = pl.program_id(0)

        @pl.when(b == 0)
        def _prep():
            copies = [
                pltpu.make_async_copy(x_hbm, x_s, sems.at[0]),
                pltpu.make_async_copy(rel_hbm, rel_s, sems.at[1]),
                pltpu.make_async_copy(w0_hbm, w0_s, sems.at[2]),
                pltpu.make_async_copy(b0_hbm, b0_s, sems.at[3]),
                pltpu.make_async_copy(w1_hbm, w1_s, sems.at[4]),
                pltpu.make_async_copy(b1_hbm, b1_s, sems.at[5]),
            ]
            for c in copies:
                c.start()
            for c in copies:
                c.wait()
            rel_c = rel_s[...].astype(_CD)
            for w_s, wx_s, relp_s, din in ((w0_s, wx0_s, relp0_s, x_s.shape[2]),
                                           (w1_s, wx1_s, relp1_s, wx0_s.shape[1] // R)):
                D = wx_s.shape[1] // R
                K = din + L
                for r in range(R):
                    wx_r = w_s[:, r * K:r * K + din].astype(_CD)
                    wx_s[:, r * D:(r + 1) * D] = wx_r.T
                    wrel_r = w_s[:, r * K + din:(r + 1) * K].astype(_CD)
                    relp_s[:, r * D:(r + 1) * D] = jax.lax.dot_general(
                        rel_c[:, r, :], wrel_r, _NT,
                        preferred_element_type=jnp.float32)

        adj_c = [a_ref[0, 0].astype(_CD) for a_ref in adj_refs]

        h = x_s[b]
        for wx_s, relp_s, b_s in ((wx0_s, relp0_s, b0_s),
                                  (wx1_s, relp1_s, b1_s)):
            D = b_s.shape[1]
            y = jnp.dot(h.astype(_CD), wx_s[...],
                        preferred_element_type=jnp.float32)
            y = (y + relp_s[pl.ds(b, 1), :]).astype(_CD)
            acc = jnp.dot(adj_c[0], y[:, :D],
                          preferred_element_type=jnp.float32)
            for r in range(1, R):
                acc += jnp.dot(adj_c[r], y[:, r * D:(r + 1) * D],
                               preferred_element_type=jnp.float32)
            h = jnp.maximum(acc + b_s[...], 0.0)
        out_ref[0] = h
    return body


def kernel(node_features, relation_features, adj, w0, b0, w1, b1):
    B, N, Din = node_features.shape
    _, R, L = relation_features.shape
    D0, D1 = w0.shape[0], w1.shape[0]

    adj_specs = [
        pl.BlockSpec((1, 1, N, N), (lambda b, rr=r: (b, rr, 0, 0)))
        for r in range(R)
    ]
    any_spec = pl.BlockSpec(memory_space=pl.ANY)
    return pl.pallas_call(
        _make_body(R, L),
        out_shape=jax.ShapeDtypeStruct((B, N, D1), node_features.dtype),
        grid=(B,),
        in_specs=[any_spec] + adj_specs + [any_spec] * 5,
        out_specs=pl.BlockSpec((1, N, D1), lambda b: (b, 0, 0)),
        scratch_shapes=[
            pltpu.VMEM((B, N, Din), jnp.float32),
            pltpu.VMEM((B, R, L), jnp.float32),
            pltpu.VMEM(w0.shape, jnp.float32),
            pltpu.VMEM((1, D0), jnp.float32),
            pltpu.VMEM(w1.shape, jnp.float32),
            pltpu.VMEM((1, D1), jnp.float32),
            pltpu.VMEM((Din, R * D0), _CD),
            pltpu.VMEM((D0, R * D1), _CD),
            pltpu.VMEM((B, R * D0), jnp.float32),
            pltpu.VMEM((B, R * D1), jnp.float32),
            pltpu.SemaphoreType.DMA((6,)),
        ],
        compiler_params=pltpu.CompilerParams(
            dimension_semantics=("arbitrary",),
            vmem_limit_bytes=int((64 << 20) * 0.75)),
    )(node_features, *([adj] * R), relation_features,
      w0, b0.reshape(1, D0), w1, b1.reshape(1, D1))


# speedup vs baseline: 1.0024x; 1.0024x over previous
import jax
import jax.numpy as jnp
from jax.experimental import pallas as pl
from jax.experimental.pallas import tpu as pltpu

_CD = jnp.bfloat16
_NT = (((1,), (1,)), ((), ()))


def _make_body(R, L):
    def body(*refs):
        x_hbm = refs[0]
        adj_refs = refs[1:1 + R]
        rel_hbm, w0_hbm, b0_hbm, w1_hbm, b1_hbm = refs[1 + R:6 + R]
        out_ref = refs[6 + R]
        (x_s, rel_s, w0_s, b0_s, w1_s, b1_s,
         wx0_s, wx1_s, relp0_s, relp1_s, sems) = refs[7 + R:]
        b = pl.program_id(0)

        @pl.when(b == 0)
        def _prep():
            copies = [
                pltpu.make_async_copy(x_hbm, x_s, sems.at[0]),
                pltpu.make_async_copy(rel_hbm, rel_s, sems.at[1]),
                pltpu.make_async_copy(w0_hbm, w0_s, sems.at[2]),
                pltpu.make_async_copy(b0_hbm, b0_s, sems.at[3]),
                pltpu.make_async_copy(w1_hbm, w1_s, sems.at[4]),
                pltpu.make_async_copy(b1_hbm, b1_s, sems.at[5]),
            ]
            for c in copies:
                c.start()
            for c in copies:
                c.wait()
            rel_c = rel_s[...].astype(_CD)
            for w_s, wx_s, relp_s, din in ((w0_s, wx0_s, relp0_s, x_s.shape[2]),
                                           (w1_s, wx1_s, relp1_s, wx0_s.shape[1] // R)):
                D = wx_s.shape[1] // R
                K = din + L
                for r in range(R):
                    wx_r = w_s[:, r * K:r * K + din].astype(_CD)
                    wx_s[:, r * D:(r + 1) * D] = wx_r.T
                    wrel_r = w_s[:, r * K + din:(r + 1) * K].astype(_CD)
                    relp_s[:, r * D:(r + 1) * D] = jax.lax.dot_general(
                        rel_c[:, r, :], wrel_r, _NT,
                        preferred_element_type=jnp.float32)

        adj_c = [a_ref[0, 0].astype(_CD) for a_ref in adj_refs]

        h = x_s[b]
        for wx_s, relp_s, b_s in ((wx0_s, relp0_s, b0_s),
                                  (wx1_s, relp1_s, b1_s)):
            D = b_s.shape[1]
            y = jnp.dot(h.astype(_CD), wx_s[...],
                        preferred_element_type=jnp.float32)
            y = (y + relp_s[pl.ds(b, 1), :]).astype(_CD)
            acc = jnp.dot(adj_c[0], y[:, :D],
                          preferred_element_type=jnp.float32)
            for r in range(1, R):
                acc += jnp.dot(adj_c[r], y[:, r * D:(r + 1) * D],
                               preferred_element_type=jnp.float32)
            h = jnp.maximum(acc + b_s[...], 0.0)
        out_ref[0] = h
    return body


def kernel(node_features, relation_features, adj, w0, b0, w1, b1):
    B, N, Din = node_features.shape
    _, R, L = relation_features.shape
    D0, D1 = w0.shape[0], w1.shape[0]

    adj_specs = [
        pl.BlockSpec((1, 1, N, N), (lambda b, rr=r: (b, rr, 0, 0)))
        for r in range(R)
    ]
    any_spec = pl.BlockSpec(memory_space=pltpu.MemorySpace.HBM)
    return pl.pallas_call(
        _make_body(R, L),
        out_shape=jax.ShapeDtypeStruct((B, N, D1), node_features.dtype),
        grid=(B,),
        in_specs=[any_spec] + adj_specs + [any_spec] * 5,
        out_specs=pl.BlockSpec((1, N, D1), lambda b: (b, 0, 0)),
        scratch_shapes=[
            pltpu.VMEM((B, N, Din), jnp.float32),
            pltpu.VMEM((B, R, L), jnp.float32),
            pltpu.VMEM(w0.shape, jnp.float32),
            pltpu.VMEM((1, D0), jnp.float32),
            pltpu.VMEM(w1.shape, jnp.float32),
            pltpu.VMEM((1, D1), jnp.float32),
            pltpu.VMEM((Din, R * D0), _CD),
            pltpu.VMEM((D0, R * D1), _CD),
            pltpu.VMEM((B, R * D0), jnp.float32),
            pltpu.VMEM((B, R * D1), jnp.float32),
            pltpu.SemaphoreType.DMA((6,)),
        ],
        compiler_params=pltpu.CompilerParams(
            dimension_semantics=("arbitrary",),
            vmem_limit_bytes=int((64 << 20) * 0.75)),
    )(node_features, *([adj] * R), relation_features,
      w0, b0.reshape(1, D0), w1, b1.reshape(1, D1))


# speedup vs baseline: 1.1289x; 1.1262x over previous
import jax
import jax.numpy as jnp
from jax.experimental import pallas as pl
from jax.experimental.pallas import tpu as pltpu

_CD = jnp.bfloat16


def _make_body(R, B, D0, D1):
    def body(x_ref, *refs):
        adj_refs = refs[:R]
        pk_ref = refs[R]
        out_ref = refs[R + 1]
        din = x_ref.shape[2]
        r0, r1 = din + D0, din + D0 + B
        rb = r1 + B
        b = pl.program_id(0)

        adj_c = [a_ref[0, 0].astype(_CD) for a_ref in adj_refs]

        h = x_ref[0]
        for w_lo, w_hi, rp_lo, b_lo, D in ((0, din, r0, 0, D0),
                                           (din, din + D0, r1, D0, D1)):
            wx = pk_ref[w_lo:w_hi, :].astype(_CD)
            y = jnp.dot(h.astype(_CD), wx, preferred_element_type=jnp.float32)
            y = (y + pk_ref[pl.ds(rp_lo + b, 1), :]).astype(_CD)
            acc = jnp.dot(adj_c[0], y[:, :D], preferred_element_type=jnp.float32)
            for r in range(1, R):
                acc += jnp.dot(adj_c[r], y[:, r * D:(r + 1) * D],
                               preferred_element_type=jnp.float32)
            bias = pk_ref[rb:rb + 1, b_lo:b_lo + D]
            h = jnp.maximum(acc + bias, 0.0)
        out_ref[0] = h
    return body


def _prep_layer(w, rel, in_dim):
    B, R, L = rel.shape
    Dout = w.shape[0]
    w_all = jnp.transpose(w).reshape(R, in_dim + L, Dout)
    wx = jnp.transpose(w_all[:, :in_dim, :], (1, 0, 2)).reshape(in_dim, R * Dout)
    relp = jnp.einsum("brl,rld->brd", rel, w_all[:, in_dim:, :])
    return wx, relp.reshape(B, R * Dout), Dout


def kernel(node_features, relation_features, adj, w0, b0, w1, b1):
    B, N, Din = node_features.shape
    _, R, _ = relation_features.shape

    wx0, relp0, D0 = _prep_layer(w0, relation_features, Din)
    wx1, relp1, D1 = _prep_layer(w1, relation_features, D0)
    W = R * max(D0, D1)

    def pad_w(a):
        return jnp.pad(a, ((0, 0), (0, W - a.shape[1])))

    bias_row = jnp.concatenate(
        [b0, b1, jnp.zeros((W - D0 - D1,), jnp.float32)])[None, :]
    rows = Din + D0 + 2 * B + 1
    packed = jnp.concatenate(
        [pad_w(wx0), pad_w(wx1), pad_w(relp0), pad_w(relp1), bias_row,
         jnp.zeros(((-rows) % 8, W), jnp.float32)], axis=0)

    adj_specs = [
        pl.BlockSpec((1, 1, N, N), (lambda b, rr=r: (b, rr, 0, 0)))
        for r in range(R)
    ]
    return pl.pallas_call(
        _make_body(R, B, D0, D1),
        out_shape=jax.ShapeDtypeStruct((B, N, D1), node_features.dtype),
        grid=(B,),
        in_specs=[pl.BlockSpec((1, N, Din), lambda b: (b, 0, 0))] + adj_specs + [
            pl.BlockSpec(packed.shape, lambda b: (0, 0)),
        ],
        out_specs=pl.BlockSpec((1, N, D1), lambda b: (b, 0, 0)),
        compiler_params=pltpu.CompilerParams(
            dimension_semantics=("arbitrary",),
            vmem_limit_bytes=int((64 << 20) * 0.75)),
    )(node_features, *([adj] * R), packed)
